# baseline (speedup 1.0000x reference)
import jax
import jax.numpy as jnp
from jax import lax
from jax.experimental import pallas as pl
from jax.experimental.pallas import tpu as pltpu
from jax.experimental.pallas import tpu_sc as plsc

VOCAB = 100000
DIM = 300
DIMP = 384
B = 1024
L = 50

NW = 32
TOTAL = B * L
CHUNK = 80
CH_PER_W = TOTAL // (NW * CHUNK)
ROWS_PER_W = TOTAL // NW


_TP_COLS = 512


def _tp_body(tt_ref, o_ref):
    o_ref[:, :DIM] = tt_ref[...].T
    o_ref[:, DIM:] = jnp.zeros((_TP_COLS, DIMP - DIM), jnp.float32)


_tc_transpose_pad = pl.pallas_call(
    _tp_body,
    grid=(pl.cdiv(VOCAB, _TP_COLS),),
    in_specs=[pl.BlockSpec((DIM, _TP_COLS), lambda i: (0, i))],
    out_specs=pl.BlockSpec((_TP_COLS, DIMP), lambda i: (i, 0)),
    out_shape=jax.ShapeDtypeStruct((VOCAB, DIMP), jnp.float32),
)


def _sc_body(table, src_idx, out, sidx_v, rows_a, rows_b, ga, gb, wa, wb):
    wid = lax.axis_index("s") * 2 + lax.axis_index("c")
    pltpu.sync_copy(src_idx.at[wid], sidx_v)

    def gather(idx_v, j, buf, sem):
        return pltpu.make_async_copy(table.at[idx_v.at[j]], buf, sem)

    def write(out, j, buf, sem):
        return pltpu.make_async_copy(
            buf, out.at[pl.ds(wid * ROWS_PER_W + j * CHUNK, CHUNK)], sem)

    def run(idx_v, out):
        gather(idx_v, 0, rows_a, ga).start()

        def body(j2, carry):
            a = 2 * j2
            gather(idx_v, a, rows_a, ga).wait()

            @pl.when(j2 > 0)
            def _():
                write(out, a - 1, rows_b, wb).wait()

            gather(idx_v, a + 1, rows_b, gb).start()
            write(out, a, rows_a, wa).start()
            gather(idx_v, a + 1, rows_b, gb).wait()
            write(out, a, rows_a, wa).wait()

            @pl.when(j2 < CH_PER_W // 2 - 1)
            def _():
                gather(idx_v, a + 2, rows_a, ga).start()

            write(out, a + 1, rows_b, wb).start()
            return carry

        lax.fori_loop(0, CH_PER_W // 2, body, 0, unroll=False)
        write(out, CH_PER_W - 1, rows_b, wb).wait()

    run(sidx_v, out)


_sc_gather = pl.kernel(
    _sc_body,
    out_type=jax.ShapeDtypeStruct((TOTAL, DIMP), jnp.float32),
    mesh=plsc.VectorSubcoreMesh(core_axis_name="c", subcore_axis_name="s"),
    scratch_types=[
        pltpu.VMEM((CH_PER_W, CHUNK), jnp.int32),
        pltpu.VMEM((CHUNK, DIMP), jnp.float32),
        pltpu.VMEM((CHUNK, DIMP), jnp.float32),
        pltpu.SemaphoreType.DMA,
        pltpu.SemaphoreType.DMA,
        pltpu.SemaphoreType.DMA,
        pltpu.SemaphoreType.DMA,
    ],
)


_UNPAD_B = 8


def _unpad_body(s_ref, os_ref):
    for bb in range(_UNPAD_B):
        os_ref[bb] = s_ref[bb * L:(bb + 1) * L, :DIM]


_tc_unpad = pl.pallas_call(
    _unpad_body,
    grid=(B // _UNPAD_B,),
    in_specs=[pl.BlockSpec((_UNPAD_B * L, DIMP), lambda i: (i, 0))],
    out_specs=pl.BlockSpec((_UNPAD_B, L, DIM), lambda i: (i, 0, 0)),
    out_shape=jax.ShapeDtypeStruct((B, L, DIM), jnp.float32),
)


def kernel(src, tgt, embedding_matrix):
    table_p = _tc_transpose_pad(embedding_matrix.T)
    src3 = src.reshape(NW, CH_PER_W, CHUNK)
    tgt3 = tgt.reshape(NW, CH_PER_W, CHUNK)
    pad_src = _sc_gather(table_p, src3)
    pad_tgt = _sc_gather(table_p, tgt3)
    return (_tc_unpad(pad_src), _tc_unpad(pad_tgt))


# speedup vs baseline: 1.5448x; 1.5448x over previous
import jax
import jax.numpy as jnp
from jax import lax
from jax.experimental import pallas as pl
from jax.experimental.pallas import tpu as pltpu
from jax.experimental.pallas import tpu_sc as plsc

VOCAB = 100000
DIM = 300
DIMP = 384
B = 1024
L = 50

NW = 32
TOTAL = B * L
CHUNK = 80
CH_PER_W = TOTAL // (NW * CHUNK)
ROWS_PER_W = TOTAL // NW


_TP_COLS = 512


def _tp_body(tt_ref, o_ref):
    o_ref[:, :DIM] = tt_ref[...].T
    o_ref[:, DIM:] = jnp.zeros((_TP_COLS, DIMP - DIM), jnp.float32)


_tc_transpose_pad = pl.pallas_call(
    _tp_body,
    grid=(pl.cdiv(VOCAB, _TP_COLS),),
    in_specs=[pl.BlockSpec((DIM, _TP_COLS), lambda i: (0, i))],
    out_specs=pl.BlockSpec((_TP_COLS, DIMP), lambda i: (i, 0)),
    out_shape=jax.ShapeDtypeStruct((VOCAB, DIMP), jnp.float32),
)


def _sc_body(table, src_idx, tgt_idx, dest_idx, out_src, out_tgt,
             sidx_v, tidx_v, dest_v, rows_a, rows_b, ga, gb, wa, wb):
    wid = lax.axis_index("s") * 2 + lax.axis_index("c")
    pltpu.sync_copy(src_idx.at[wid], sidx_v)
    pltpu.sync_copy(tgt_idx.at[wid], tidx_v)
    pltpu.sync_copy(dest_idx.at[wid], dest_v)

    def gather(idx_v, j, buf, sem):
        return pltpu.make_async_copy(table.at[idx_v.at[j]], buf, sem)

    def write(out, j, buf, sem):
        return pltpu.make_async_copy(buf, out.at[dest_v.at[j]], sem)

    def run(idx_v, out):
        gather(idx_v, 0, rows_a, ga).start()

        def body(j2, carry):
            a = 2 * j2
            gather(idx_v, a, rows_a, ga).wait()

            @pl.when(j2 > 0)
            def _():
                write(out, a - 1, rows_b, wb).wait()

            gather(idx_v, a + 1, rows_b, gb).start()
            write(out, a, rows_a, wa).start()
            gather(idx_v, a + 1, rows_b, gb).wait()
            write(out, a, rows_a, wa).wait()

            @pl.when(j2 < CH_PER_W // 2 - 1)
            def _():
                gather(idx_v, a + 2, rows_a, ga).start()

            write(out, a + 1, rows_b, wb).start()
            return carry

        lax.fori_loop(0, CH_PER_W // 2, body, 0, unroll=False)
        write(out, CH_PER_W - 1, rows_b, wb).wait()

    run(sidx_v, out_src)
    run(tidx_v, out_tgt)


_sc_gather = pl.kernel(
    _sc_body,
    out_type=(
        jax.ShapeDtypeStruct((TOTAL, DIMP), jnp.float32),
        jax.ShapeDtypeStruct((TOTAL, DIMP), jnp.float32),
    ),
    mesh=plsc.VectorSubcoreMesh(core_axis_name="c", subcore_axis_name="s"),
    scratch_types=[
        pltpu.VMEM((CH_PER_W, CHUNK), jnp.int32),
        pltpu.VMEM((CH_PER_W, CHUNK), jnp.int32),
        pltpu.VMEM((CH_PER_W, CHUNK), jnp.int32),
        pltpu.VMEM((CHUNK, DIMP), jnp.float32),
        pltpu.VMEM((CHUNK, DIMP), jnp.float32),
        pltpu.SemaphoreType.DMA,
        pltpu.SemaphoreType.DMA,
        pltpu.SemaphoreType.DMA,
        pltpu.SemaphoreType.DMA,
    ],
)


def _out_body(s_ref, t_ref, os_ref, ot_ref):
    os_ref[0] = s_ref[0][:, :DIM].T
    ot_ref[0] = t_ref[0][:, :DIM].T


_tc_out = pl.pallas_call(
    _out_body,
    grid=(L,),
    in_specs=[
        pl.BlockSpec((1, B, DIMP), lambda i: (i, 0, 0)),
        pl.BlockSpec((1, B, DIMP), lambda i: (i, 0, 0)),
    ],
    out_specs=[
        pl.BlockSpec((1, DIM, B), lambda i: (i, 0, 0)),
        pl.BlockSpec((1, DIM, B), lambda i: (i, 0, 0)),
    ],
    out_shape=[
        jax.ShapeDtypeStruct((L, DIM, B), jnp.float32),
        jax.ShapeDtypeStruct((L, DIM, B), jnp.float32),
    ],
)


def kernel(src, tgt, embedding_matrix):
    table_p = _tc_transpose_pad(embedding_matrix.T)
    src3 = src.reshape(NW, CH_PER_W, CHUNK)
    tgt3 = tgt.reshape(NW, CH_PER_W, CHUNK)
    g = jnp.arange(TOTAL, dtype=jnp.int32)
    dest3 = ((g % L) * B + g // L).reshape(NW, CH_PER_W, CHUNK)
    pad_src, pad_tgt = _sc_gather(table_p, src3, tgt3, dest3)
    ys, yt = _tc_out(pad_src.reshape(L, B, DIMP), pad_tgt.reshape(L, B, DIMP))
    return (jnp.transpose(ys, (2, 0, 1)), jnp.transpose(yt, (2, 0, 1)))


# speedup vs baseline: 1.7719x; 1.1470x over previous
import jax
import jax.numpy as jnp
from jax import lax
from jax.experimental import pallas as pl
from jax.experimental.pallas import tpu as pltpu
from jax.experimental.pallas import tpu_sc as plsc

VOCAB = 100000
DIM = 300
DIMP = 384
B = 1024
L = 50

NW = 32
TOTAL = B * L
CHUNK = 50
CH_PER_W = TOTAL // (NW * CHUNK)
ROWS_PER_W = TOTAL // NW


_TP_COLS = 1024


def _tp_body(tt_ref, o_ref):
    o_ref[:, :DIM] = tt_ref[...].T
    o_ref[:, DIM:] = jnp.zeros((_TP_COLS, DIMP - DIM), jnp.float32)


_tc_transpose_pad = pl.pallas_call(
    _tp_body,
    grid=(pl.cdiv(VOCAB, _TP_COLS),),
    in_specs=[pl.BlockSpec((DIM, _TP_COLS), lambda i: (0, i))],
    out_specs=pl.BlockSpec((_TP_COLS, DIMP), lambda i: (i, 0)),
    out_shape=jax.ShapeDtypeStruct((VOCAB, DIMP), jnp.float32),
)


_NBUF = 4


def _sc_body(table, src_idx, tgt_idx, dest_idx, out_src, out_tgt,
             sidx_v, tidx_v, dest_v, rows0, rows1, rows2, rows3, gsem, wsem):
    wid = lax.axis_index("s") * 2 + lax.axis_index("c")
    rows = [rows0, rows1, rows2, rows3]
    pltpu.sync_copy(src_idx.at[wid], sidx_v)
    pltpu.sync_copy(tgt_idx.at[wid], tidx_v)
    pltpu.sync_copy(dest_idx.at[wid], dest_v)

    def gather(idx_v, j, k):
        return pltpu.make_async_copy(
            table.at[idx_v.at[j]], rows[k], gsem.at[k])

    def write(out, j, k):
        return pltpu.make_async_copy(
            rows[k], out.at[dest_v.at[j]], wsem.at[k])

    def run(idx_v, out):
        for k in range(_NBUF - 1):
            gather(idx_v, k, k).start()

        def body(j4, carry):
            for k in range(_NBUF):
                c = _NBUF * j4 + k
                km1 = (k - 1) % _NBUF
                gather(idx_v, c, k).wait()
                write(out, c, k).start()
                nxt = c + _NBUF - 1

                @pl.when(jnp.logical_and(c >= 1, nxt < CH_PER_W))
                def _():
                    write(out, c - 1, km1).wait()

                @pl.when(nxt < CH_PER_W)
                def _():
                    gather(idx_v, nxt, km1).start()

            return carry

        lax.fori_loop(0, CH_PER_W // _NBUF, body, 0, unroll=False)
        for k in range(_NBUF):
            write(out, CH_PER_W - _NBUF + k, k).wait()

    run(sidx_v, out_src)
    run(tidx_v, out_tgt)


_sc_gather = pl.kernel(
    _sc_body,
    out_type=(
        jax.ShapeDtypeStruct((TOTAL, DIMP), jnp.float32),
        jax.ShapeDtypeStruct((TOTAL, DIMP), jnp.float32),
    ),
    mesh=plsc.VectorSubcoreMesh(core_axis_name="c", subcore_axis_name="s"),
    scratch_types=[
        pltpu.VMEM((CH_PER_W, CHUNK), jnp.int32),
        pltpu.VMEM((CH_PER_W, CHUNK), jnp.int32),
        pltpu.VMEM((CH_PER_W, CHUNK), jnp.int32),
        pltpu.VMEM((CHUNK, DIMP), jnp.float32),
        pltpu.VMEM((CHUNK, DIMP), jnp.float32),
        pltpu.VMEM((CHUNK, DIMP), jnp.float32),
        pltpu.VMEM((CHUNK, DIMP), jnp.float32),
        pltpu.SemaphoreType.DMA((_NBUF,)),
        pltpu.SemaphoreType.DMA((_NBUF,)),
    ],
)


def _out_body(s_ref, t_ref, os_ref, ot_ref):
    os_ref[0] = s_ref[0][:, :DIM].T
    ot_ref[0] = t_ref[0][:, :DIM].T


_tc_out = pl.pallas_call(
    _out_body,
    grid=(L,),
    in_specs=[
        pl.BlockSpec((1, B, DIMP), lambda i: (i, 0, 0)),
        pl.BlockSpec((1, B, DIMP), lambda i: (i, 0, 0)),
    ],
    out_specs=[
        pl.BlockSpec((1, DIM, B), lambda i: (i, 0, 0)),
        pl.BlockSpec((1, DIM, B), lambda i: (i, 0, 0)),
    ],
    out_shape=[
        jax.ShapeDtypeStruct((L, DIM, B), jnp.float32),
        jax.ShapeDtypeStruct((L, DIM, B), jnp.float32),
    ],
)


def kernel(src, tgt, embedding_matrix):
    table_p = _tc_transpose_pad(embedding_matrix.T)
    src3 = src.reshape(NW, CH_PER_W, CHUNK)
    tgt3 = tgt.reshape(NW, CH_PER_W, CHUNK)
    g = jnp.arange(TOTAL, dtype=jnp.int32)
    dest3 = ((g % L) * B + g // L).reshape(NW, CH_PER_W, CHUNK)
    pad_src, pad_tgt = _sc_gather(table_p, src3, tgt3, dest3)
    ys, yt = _tc_out(pad_src.reshape(L, B, DIMP), pad_tgt.reshape(L, B, DIMP))
    return (jnp.transpose(ys, (2, 0, 1)), jnp.transpose(yt, (2, 0, 1)))
